# baseline (speedup 1.0000x reference)
import functools

import jax
import jax.numpy as jnp
from jax import lax
from jax.experimental import pallas as pl
from jax.experimental.pallas import tpu as pltpu
from jax.experimental.pallas import tpu_sc as plsc

_SC_PARAMS = pltpu.CompilerParams(needs_layout_passes=False)
_NC = 2
_NS = 16
_LANES = 128


def _mesh():
    return plsc.VectorSubcoreMesh(core_axis_name="c", subcore_axis_name="s")


def _make_deg(n_acc, e_pad):
    per = e_pad // (_NC * _NS)
    n_vec = per // 16

    @functools.partial(
        pl.kernel,
        out_type=jax.ShapeDtypeStruct((_NC * _NS, n_acc), jnp.float32),
        mesh=_mesh(),
        scratch_types=[
            pltpu.VMEM((per,), jnp.int32),
            pltpu.VMEM((per,), jnp.float32),
            pltpu.VMEM((n_acc,), jnp.float32),
        ],
        compiler_params=_SC_PARAMS,
    )
    def deg_kernel(dst_hbm, w_hbm, out_hbm, dst_v, w_v, acc):
        cid = lax.axis_index("c")
        sid = lax.axis_index("s")
        wid = sid * _NC + cid
        base = wid * per
        pltpu.sync_copy(dst_hbm.at[pl.ds(base, per)], dst_v)
        pltpu.sync_copy(w_hbm.at[pl.ds(base, per)], w_v)

        zeros16 = jnp.zeros((16,), jnp.float32)

        def zero(i, _):
            acc[pl.ds(i * 16, 16)] = zeros16
            return 0

        lax.fori_loop(0, n_acc // 16, zero, 0)

        def add(i, _):
            idx16 = dst_v[pl.ds(i * 16, 16)]
            w16 = w_v[pl.ds(i * 16, 16)]
            plsc.addupdate_scatter(acc, [idx16], w16)
            return 0

        lax.fori_loop(0, n_vec, add, 0)
        pltpu.sync_copy(acc, out_hbm.at[wid])

    return deg_kernel


def _make_agg(n_pad, k_chunks, feat_split):
    rows_per_sub = n_pad // _NS
    blk = 16
    assert rows_per_sub % _LANES == 0 and k_chunks % blk == 0 and blk % 2 == 0
    n_init = rows_per_sub // _LANES
    n_blk = k_chunks // blk

    @functools.partial(
        pl.kernel,
        out_type=jax.ShapeDtypeStruct((2, n_pad, _LANES), jnp.float32),
        mesh=_mesh(),
        scratch_types=[
            pltpu.VMEM((blk, _LANES), jnp.int32),
            pltpu.VMEM((blk, _LANES), jnp.int32),
            pltpu.VMEM((blk, _LANES), jnp.int32),
            pltpu.VMEM((blk, _LANES), jnp.int32),
            pltpu.VMEM((blk, _LANES), jnp.float32),
            pltpu.VMEM((blk, _LANES), jnp.float32),
            pltpu.VMEM((_LANES, _LANES), jnp.float32),
            pltpu.VMEM((_LANES, _LANES), jnp.float32),
            pltpu.VMEM_SHARED((n_pad, _LANES), jnp.float32),
            pltpu.SemaphoreType.DMA,
            pltpu.SemaphoreType.DMA,
            pltpu.SemaphoreType.DMA,
            pltpu.SemaphoreType.DMA,
            pltpu.SemaphoreType.DMA,
            pltpu.SemaphoreType.DMA,
        ],
        compiler_params=_SC_PARAMS,
    )
    def agg_kernel(h_hbm, src_hbm, dst_hbm, w_hbm, out_hbm,
                   src_a, src_b, dst_a, dst_b, w_a, w_b, rows_a, rows_b,
                   sh, sem_a, sem_b, sem_pa, sem_pb, sem_sa, sem_sb):
        cid = lax.axis_index("c")
        sid = lax.axis_index("s")
        zeros16 = jnp.zeros((16,), jnp.float32)
        pans = [(src_a, dst_a, w_a, sem_pa), (src_b, dst_b, w_b, sem_pb)]

        def h_view():
            return h_hbm.at[cid] if feat_split else h_hbm

        row0 = sid * rows_per_sub

        def zero_rows(i, _):
            for j in range(_LANES // 16):
                rows_a[i, pl.ds(j * 16, 16)] = zeros16
            return 0

        lax.fori_loop(0, _LANES, zero_rows, 0)
        for q in range(n_init):
            pltpu.sync_copy(rows_a, sh.at[pl.ds(row0 + q * _LANES, _LANES)])

        plsc.subcore_barrier()

        def scale(buf, wbuf, k):
            def body(g, _):
                w16 = wbuf[k, pl.ds(g * 16, 16)]
                for l in range(16):
                    wi = w16[l]
                    i = g * 16 + l
                    for j in range(_LANES // 16):
                        buf[i, pl.ds(j * 16, 16)] = buf[i, pl.ds(j * 16, 16)] * wi
                return 0

            lax.fori_loop(0, _LANES // 16, body, 0)

        def block_copies(p, b, bufs):
            sv, dv, wv, sem = bufs
            sl = pl.ds(b * blk, blk)
            return [(src_hbm.at[p].at[sl], sv, sem),
                    (dst_hbm.at[p].at[sl], dv, sem),
                    (w_hbm.at[p].at[sl], wv, sem)]

        def start_block(p, b, bufs):
            for s, d, sem in block_copies(p, b, bufs):
                pltpu.async_copy(s, d, sem)

        def wait_block(p, b, bufs):
            for s, d, sem in block_copies(p, b, bufs):
                pltpu.make_async_copy(s, d, sem).wait()

        def process_block(bufs):
            sv, dv, wv, _ = bufs
            pltpu.async_copy(h_view().at[sv.at[0]], rows_a, sem_a)

            def pair(t2, _):
                t = t2 * 2

                @pl.when(t > 0)
                def _wait_sb():
                    pltpu.make_async_copy(rows_b, sh.at[dv.at[t - 1]], sem_sb).wait()

                pltpu.async_copy(h_view().at[sv.at[t + 1]], rows_b, sem_b)
                pltpu.make_async_copy(h_view().at[sv.at[t]], rows_a, sem_a).wait()
                scale(rows_a, wv, t)
                pltpu.async_copy(rows_a, sh.at[dv.at[t]], sem_sa, add=True)
                pltpu.make_async_copy(h_view().at[sv.at[t + 1]], rows_b, sem_b).wait()
                scale(rows_b, wv, t + 1)
                pltpu.make_async_copy(rows_a, sh.at[dv.at[t]], sem_sa).wait()

                @pl.when(t + 2 < blk)
                def _next_a():
                    pltpu.async_copy(h_view().at[sv.at[t + 2]], rows_a, sem_a)

                pltpu.async_copy(rows_b, sh.at[dv.at[t + 1]], sem_sb, add=True)
                return 0

            lax.fori_loop(0, blk // 2, pair, 0)
            pltpu.make_async_copy(rows_b, sh.at[dv.at[blk - 1]], sem_sb).wait()

        def do_panel(p):
            start_block(p, 0, pans[0])
            wait_block(p, 0, pans[0])
            for b in range(n_blk):
                if b + 1 < n_blk:
                    start_block(p, b + 1, pans[(b + 1) % 2])
                process_block(pans[b % 2])
                if b + 1 < n_blk:
                    wait_block(p, b + 1, pans[(b + 1) % 2])

        if feat_split:
            def panel_loop(i, _):
                do_panel(sid * 2 + i)
                return 0

            lax.fori_loop(0, 2, panel_loop, 0)
        else:
            do_panel(sid * 2 + cid)

        plsc.subcore_barrier()

        for q in range(n_init):
            r = row0 + q * _LANES
            pltpu.sync_copy(sh.at[pl.ds(r, _LANES)], rows_a)
            pltpu.sync_copy(rows_a, out_hbm.at[cid].at[pl.ds(r, _LANES)])

    return agg_kernel


def _mm1_body(x_ref, w_ref, degp_ref, h3_ref, dis_ref):
    deg = jnp.sum(degp_ref[...], axis=0) + 1.0
    dis = jnp.where(deg > 0, lax.rsqrt(deg), 0.0)
    h = jnp.dot(x_ref[...], w_ref[...], preferred_element_type=jnp.float32)
    hh = dis[:, None] * h
    half = h3_ref.shape[2]
    h3_ref[0] = hh[:, :half]
    h3_ref[1] = hh[:, half:]
    dis_ref[...] = dis


def _mm2_body(a_ref, h3_ref, dis_ref, b1_ref, w2_ref, out_ref):
    dis = dis_ref[...]
    a = jnp.concatenate([a_ref[0] + h3_ref[0], a_ref[1] + h3_ref[1]], axis=1)
    z = jnp.maximum(dis[:, None] * a + b1_ref[...][None, :], 0.0)
    h2 = jnp.dot(z, w2_ref[...], preferred_element_type=jnp.float32)
    out_ref[...] = dis[:, None] * h2


def _end_body(p_ref, h2_ref, dis_ref, b2_ref, out_ref):
    s = p_ref[0] + p_ref[1] + h2_ref[...]
    out_ref[...] = dis_ref[...][:, None] * s + b2_ref[...][None, :]


def kernel(x, edge_index, edge_attr, W1, b1, W2, b2):
    n, in_ch = x.shape
    hid = W1.shape[1]
    out_ch = W2.shape[1]
    e = edge_attr.shape[0]
    assert hid == 2 * _LANES and out_ch == _LANES
    row_t = 1024
    n_pad = -(-n // row_t) * row_t

    src = edge_index[0].astype(jnp.int32)
    dst = edge_index[1].astype(jnp.int32)
    w = edge_attr.astype(jnp.float32)
    n_panels = 2 * _NS
    k_chunks = -(-e // (n_panels * _LANES))
    k_chunks += k_chunks % 2
    e_pad = n_panels * k_chunks * _LANES
    pad = e_pad - e
    srcp = jnp.pad(src, (0, pad)).reshape(n_panels, k_chunks, _LANES)
    dstp = jnp.pad(dst, (0, pad)).reshape(n_panels, k_chunks, _LANES)
    wp = jnp.pad(w, (0, pad)).reshape(n_panels, k_chunks, _LANES)

    deg_parts = _make_deg(n_pad, e_pad)(dstp.reshape(-1), wp.reshape(-1))

    xp = jnp.pad(x, ((0, n_pad - n), (0, 0)))
    grid = (n_pad // row_t,)
    h3, dis = pl.pallas_call(
        _mm1_body,
        grid=grid,
        in_specs=[
            pl.BlockSpec((row_t, in_ch), lambda r: (r, 0)),
            pl.BlockSpec((in_ch, hid), lambda r: (0, 0)),
            pl.BlockSpec((_NC * _NS, row_t), lambda r: (0, r)),
        ],
        out_specs=[
            pl.BlockSpec((2, row_t, _LANES), lambda r: (0, r, 0)),
            pl.BlockSpec((row_t,), lambda r: (r,)),
        ],
        out_shape=[
            jax.ShapeDtypeStruct((2, n_pad, _LANES), jnp.float32),
            jax.ShapeDtypeStruct((n_pad,), jnp.float32),
        ],
    )(xp, W1, deg_parts)

    agg1 = _make_agg(n_pad, k_chunks, True)(h3, srcp, dstp, wp)

    h2p = pl.pallas_call(
        _mm2_body,
        grid=grid,
        in_specs=[
            pl.BlockSpec((2, row_t, _LANES), lambda r: (0, r, 0)),
            pl.BlockSpec((2, row_t, _LANES), lambda r: (0, r, 0)),
            pl.BlockSpec((row_t,), lambda r: (r,)),
            pl.BlockSpec((hid,), lambda r: (0,)),
            pl.BlockSpec((hid, out_ch), lambda r: (0, 0)),
        ],
        out_specs=pl.BlockSpec((row_t, out_ch), lambda r: (r, 0)),
        out_shape=jax.ShapeDtypeStruct((n_pad, out_ch), jnp.float32),
    )(agg1, h3, dis, b1, W2)

    parts = _make_agg(n_pad, k_chunks, False)(h2p, srcp, dstp, wp)

    out = pl.pallas_call(
        _end_body,
        grid=grid,
        in_specs=[
            pl.BlockSpec((2, row_t, _LANES), lambda r: (0, r, 0)),
            pl.BlockSpec((row_t, out_ch), lambda r: (r, 0)),
            pl.BlockSpec((row_t,), lambda r: (r,)),
            pl.BlockSpec((out_ch,), lambda r: (0,)),
        ],
        out_specs=pl.BlockSpec((row_t, out_ch), lambda r: (r, 0)),
        out_shape=jax.ShapeDtypeStruct((n_pad, out_ch), jnp.float32),
    )(parts, h2p, dis, b2)
    return out[:n]


# speedup vs baseline: 1.1478x; 1.1478x over previous
import functools

import jax
import jax.numpy as jnp
from jax import lax
from jax.experimental import pallas as pl
from jax.experimental.pallas import tpu as pltpu
from jax.experimental.pallas import tpu_sc as plsc

_SC_PARAMS = pltpu.CompilerParams(needs_layout_passes=False)
_NC = 2
_NS = 16
_LANES = 128
_NSEM = 4


def _mesh():
    return plsc.VectorSubcoreMesh(core_axis_name="c", subcore_axis_name="s")


def _make_deg(n_acc, e_pad):
    per = e_pad // (_NC * _NS)
    n_vec = per // 16

    @functools.partial(
        pl.kernel,
        out_type=jax.ShapeDtypeStruct((_NC * _NS, n_acc), jnp.float32),
        mesh=_mesh(),
        scratch_types=[
            pltpu.VMEM((per,), jnp.int32),
            pltpu.VMEM((per,), jnp.float32),
            pltpu.VMEM((n_acc,), jnp.float32),
        ],
        compiler_params=_SC_PARAMS,
    )
    def deg_kernel(dst_hbm, w_hbm, out_hbm, dst_v, w_v, acc):
        cid = lax.axis_index("c")
        sid = lax.axis_index("s")
        wid = sid * _NC + cid
        base = wid * per
        pltpu.sync_copy(dst_hbm.at[pl.ds(base, per)], dst_v)
        pltpu.sync_copy(w_hbm.at[pl.ds(base, per)], w_v)

        zeros16 = jnp.zeros((16,), jnp.float32)

        def zero(i, _):
            acc[pl.ds(i * 16, 16)] = zeros16
            return 0

        lax.fori_loop(0, n_acc // 16, zero, 0)

        def add(i, _):
            idx16 = dst_v[pl.ds(i * 16, 16)]
            w16 = w_v[pl.ds(i * 16, 16)]
            plsc.addupdate_scatter(acc, [idx16], w16)
            return 0

        lax.fori_loop(0, n_vec, add, 0)
        pltpu.sync_copy(acc, out_hbm.at[wid])

    return deg_kernel


def _make_agg(n_pad, k_chunks, feat_split):
    rows_per_sub = n_pad // _NS
    assert rows_per_sub % _LANES == 0
    n_init = rows_per_sub // _LANES
    qrows = _LANES // _NSEM

    @functools.partial(
        pl.kernel,
        out_type=jax.ShapeDtypeStruct((2, n_pad, _LANES), jnp.float32),
        mesh=_mesh(),
        scratch_types=[
            pltpu.VMEM((k_chunks, _LANES), jnp.int32),
            pltpu.VMEM((k_chunks, _LANES), jnp.int32),
            pltpu.VMEM((k_chunks, _LANES), jnp.float32),
            pltpu.VMEM((_LANES, _LANES), jnp.float32),
            pltpu.VMEM_SHARED((n_pad, _LANES), jnp.float32),
            [pltpu.SemaphoreType.DMA] * _NSEM,
        ],
        compiler_params=_SC_PARAMS,
    )
    def agg_kernel(h_hbm, src_hbm, dst_hbm, w_hbm, out_hbm,
                   src_v, dst_v, w_v, rows, sh, sems):
        cid = lax.axis_index("c")
        sid = lax.axis_index("s")
        zeros16 = jnp.zeros((16,), jnp.float32)

        def h_view():
            return h_hbm.at[cid] if feat_split else h_hbm

        row0 = sid * rows_per_sub

        def zero_rows(i, _):
            for j in range(_LANES // 16):
                rows[i, pl.ds(j * 16, 16)] = zeros16
            return 0

        lax.fori_loop(0, _LANES, zero_rows, 0)
        for q in range(n_init):
            pltpu.sync_copy(rows, sh.at[pl.ds(row0 + q * _LANES, _LANES)])

        plsc.subcore_barrier()

        def chunk(k, _):
            for j in range(_NSEM):
                pltpu.async_copy(
                    h_view().at[src_v.at[k, pl.ds(j * qrows, qrows)]],
                    rows.at[pl.ds(j * qrows, qrows)], sems[j])
            for j in range(_NSEM):
                pltpu.make_async_copy(
                    h_view().at[src_v.at[k, pl.ds(j * qrows, qrows)]],
                    rows.at[pl.ds(j * qrows, qrows)], sems[j]).wait()
                for g in range(qrows // 16):
                    w16 = w_v[k, pl.ds(j * qrows + g * 16, 16)]
                    for l in range(16):
                        wi = w16[l]
                        i = j * qrows + g * 16 + l
                        for f in range(_LANES // 16):
                            rows[i, pl.ds(f * 16, 16)] = rows[i, pl.ds(f * 16, 16)] * wi
            pltpu.sync_copy(rows, sh.at[dst_v.at[k]], add=True)
            return 0

        def do_panel(p):
            pltpu.sync_copy(src_hbm.at[p], src_v)
            pltpu.sync_copy(dst_hbm.at[p], dst_v)
            pltpu.sync_copy(w_hbm.at[p], w_v)
            lax.fori_loop(0, k_chunks, chunk, 0)

        if feat_split:
            def panel_loop(i, _):
                do_panel(sid * 2 + i)
                return 0

            lax.fori_loop(0, 2, panel_loop, 0)
        else:
            do_panel(sid * 2 + cid)

        plsc.subcore_barrier()

        for q in range(n_init):
            r = row0 + q * _LANES
            pltpu.sync_copy(sh.at[pl.ds(r, _LANES)], rows)
            pltpu.sync_copy(rows, out_hbm.at[cid].at[pl.ds(r, _LANES)])

    return agg_kernel


def _mm1_body(x_ref, w_ref, degp_ref, h3_ref, dis_ref):
    deg = jnp.sum(degp_ref[...], axis=0) + 1.0
    dis = jnp.where(deg > 0, lax.rsqrt(deg), 0.0)
    h = jnp.dot(x_ref[...], w_ref[...], preferred_element_type=jnp.float32)
    hh = dis[:, None] * h
    half = h3_ref.shape[2]
    h3_ref[0] = hh[:, :half]
    h3_ref[1] = hh[:, half:]
    dis_ref[...] = dis


def _mm2_body(a_ref, h3_ref, dis_ref, b1_ref, w2_ref, out_ref):
    dis = dis_ref[...]
    a = jnp.concatenate([a_ref[0] + h3_ref[0], a_ref[1] + h3_ref[1]], axis=1)
    z = jnp.maximum(dis[:, None] * a + b1_ref[...][None, :], 0.0)
    h2 = jnp.dot(z, w2_ref[...], preferred_element_type=jnp.float32)
    out_ref[...] = dis[:, None] * h2


def _end_body(p_ref, h2_ref, dis_ref, b2_ref, out_ref):
    s = p_ref[0] + p_ref[1] + h2_ref[...]
    out_ref[...] = dis_ref[...][:, None] * s + b2_ref[...][None, :]


def kernel(x, edge_index, edge_attr, W1, b1, W2, b2):
    n, in_ch = x.shape
    hid = W1.shape[1]
    out_ch = W2.shape[1]
    e = edge_attr.shape[0]
    assert hid == 2 * _LANES and out_ch == _LANES
    row_t = 1024
    n_pad = -(-n // row_t) * row_t

    src = edge_index[0].astype(jnp.int32)
    dst = edge_index[1].astype(jnp.int32)
    w = edge_attr.astype(jnp.float32)
    n_panels = 2 * _NS
    k_chunks = -(-e // (n_panels * _LANES))
    e_pad = n_panels * k_chunks * _LANES
    pad = e_pad - e
    srcp = jnp.pad(src, (0, pad)).reshape(n_panels, k_chunks, _LANES)
    dstp = jnp.pad(dst, (0, pad)).reshape(n_panels, k_chunks, _LANES)
    wp = jnp.pad(w, (0, pad)).reshape(n_panels, k_chunks, _LANES)

    deg_parts = _make_deg(n_pad, e_pad)(dstp.reshape(-1), wp.reshape(-1))

    xp = jnp.pad(x, ((0, n_pad - n), (0, 0)))
    grid = (n_pad // row_t,)
    h3, dis = pl.pallas_call(
        _mm1_body,
        grid=grid,
        in_specs=[
            pl.BlockSpec((row_t, in_ch), lambda r: (r, 0)),
            pl.BlockSpec((in_ch, hid), lambda r: (0, 0)),
            pl.BlockSpec((_NC * _NS, row_t), lambda r: (0, r)),
        ],
        out_specs=[
            pl.BlockSpec((2, row_t, _LANES), lambda r: (0, r, 0)),
            pl.BlockSpec((row_t,), lambda r: (r,)),
        ],
        out_shape=[
            jax.ShapeDtypeStruct((2, n_pad, _LANES), jnp.float32),
            jax.ShapeDtypeStruct((n_pad,), jnp.float32),
        ],
    )(xp, W1, deg_parts)

    agg1 = _make_agg(n_pad, k_chunks, True)(h3, srcp, dstp, wp)

    h2p = pl.pallas_call(
        _mm2_body,
        grid=grid,
        in_specs=[
            pl.BlockSpec((2, row_t, _LANES), lambda r: (0, r, 0)),
            pl.BlockSpec((2, row_t, _LANES), lambda r: (0, r, 0)),
            pl.BlockSpec((row_t,), lambda r: (r,)),
            pl.BlockSpec((hid,), lambda r: (0,)),
            pl.BlockSpec((hid, out_ch), lambda r: (0, 0)),
        ],
        out_specs=pl.BlockSpec((row_t, out_ch), lambda r: (r, 0)),
        out_shape=jax.ShapeDtypeStruct((n_pad, out_ch), jnp.float32),
    )(agg1, h3, dis, b1, W2)

    parts = _make_agg(n_pad, k_chunks, False)(h2p, srcp, dstp, wp)

    out = pl.pallas_call(
        _end_body,
        grid=grid,
        in_specs=[
            pl.BlockSpec((2, row_t, _LANES), lambda r: (0, r, 0)),
            pl.BlockSpec((row_t, out_ch), lambda r: (r, 0)),
            pl.BlockSpec((row_t,), lambda r: (r,)),
            pl.BlockSpec((out_ch,), lambda r: (0,)),
        ],
        out_specs=pl.BlockSpec((row_t, out_ch), lambda r: (r, 0)),
        out_shape=jax.ShapeDtypeStruct((n_pad, out_ch), jnp.float32),
    )(parts, h2p, dis, b2)
    return out[:n]


# speedup vs baseline: 1.1878x; 1.0348x over previous
import functools

import jax
import jax.numpy as jnp
from jax import lax
from jax.experimental import pallas as pl
from jax.experimental.pallas import tpu as pltpu
from jax.experimental.pallas import tpu_sc as plsc

_SC_PARAMS = pltpu.CompilerParams(needs_layout_passes=False)
_NC = 2
_NS = 16
_LANES = 128
_NSEM = 8


def _mesh():
    return plsc.VectorSubcoreMesh(core_axis_name="c", subcore_axis_name="s")


def _make_deg(n_acc, e_pad):
    per = e_pad // (_NC * _NS)
    n_vec = per // 16

    @functools.partial(
        pl.kernel,
        out_type=jax.ShapeDtypeStruct((_NC * _NS, n_acc), jnp.float32),
        mesh=_mesh(),
        scratch_types=[
            pltpu.VMEM((per,), jnp.int32),
            pltpu.VMEM((per,), jnp.float32),
            pltpu.VMEM((n_acc,), jnp.float32),
        ],
        compiler_params=_SC_PARAMS,
    )
    def deg_kernel(dst_hbm, w_hbm, out_hbm, dst_v, w_v, acc):
        cid = lax.axis_index("c")
        sid = lax.axis_index("s")
        wid = sid * _NC + cid
        base = wid * per
        pltpu.sync_copy(dst_hbm.at[pl.ds(base, per)], dst_v)
        pltpu.sync_copy(w_hbm.at[pl.ds(base, per)], w_v)

        zeros16 = jnp.zeros((16,), jnp.float32)

        def zero(i, _):
            acc[pl.ds(i * 16, 16)] = zeros16
            return 0

        lax.fori_loop(0, n_acc // 16, zero, 0)

        def add(i, _):
            idx16 = dst_v[pl.ds(i * 16, 16)]
            w16 = w_v[pl.ds(i * 16, 16)]
            plsc.addupdate_scatter(acc, [idx16], w16)
            return 0

        lax.fori_loop(0, n_vec, add, 0)
        pltpu.sync_copy(acc, out_hbm.at[wid])

    return deg_kernel


def _make_agg(n_pad, k_chunks, feat_split):
    rows_per_sub = n_pad // _NS
    assert rows_per_sub % _LANES == 0
    n_init = rows_per_sub // _LANES
    qrows = _LANES // _NSEM

    @functools.partial(
        pl.kernel,
        out_type=jax.ShapeDtypeStruct((2, n_pad, _LANES), jnp.float32),
        mesh=_mesh(),
        scratch_types=[
            pltpu.VMEM((k_chunks, _LANES), jnp.int32),
            pltpu.VMEM((k_chunks, _LANES), jnp.int32),
            pltpu.VMEM((k_chunks, _LANES), jnp.float32),
            pltpu.VMEM((_LANES, _LANES), jnp.float32),
            pltpu.VMEM_SHARED((n_pad, _LANES), jnp.float32),
            [pltpu.SemaphoreType.DMA] * _NSEM,
        ],
        compiler_params=_SC_PARAMS,
    )
    def agg_kernel(h_hbm, src_hbm, dst_hbm, w_hbm, out_hbm,
                   src_v, dst_v, w_v, rows, sh, sems):
        cid = lax.axis_index("c")
        sid = lax.axis_index("s")
        zeros16 = jnp.zeros((16,), jnp.float32)

        def h_view():
            return h_hbm.at[cid] if feat_split else h_hbm

        row0 = sid * rows_per_sub

        def zero_rows(i, _):
            for j in range(_LANES // 16):
                rows[i, pl.ds(j * 16, 16)] = zeros16
            return 0

        lax.fori_loop(0, _LANES, zero_rows, 0)
        for q in range(n_init):
            pltpu.sync_copy(rows, sh.at[pl.ds(row0 + q * _LANES, _LANES)])

        plsc.subcore_barrier()

        def chunk(k, _):
            for j in range(_NSEM):
                pltpu.async_copy(
                    h_view().at[src_v.at[k, pl.ds(j * qrows, qrows)]],
                    rows.at[pl.ds(j * qrows, qrows)], sems[j])
            for j in range(_NSEM):
                pltpu.make_async_copy(
                    h_view().at[src_v.at[k, pl.ds(j * qrows, qrows)]],
                    rows.at[pl.ds(j * qrows, qrows)], sems[j]).wait()
                for g in range(qrows // 16):
                    w16 = w_v[k, pl.ds(j * qrows + g * 16, 16)]
                    for l in range(16):
                        wi = w16[l]
                        i = j * qrows + g * 16 + l
                        for f in range(_LANES // 16):
                            rows[i, pl.ds(f * 16, 16)] = rows[i, pl.ds(f * 16, 16)] * wi
            pltpu.sync_copy(rows, sh.at[dst_v.at[k]], add=True)
            return 0

        def do_panel(p):
            pltpu.sync_copy(src_hbm.at[p], src_v)
            pltpu.sync_copy(dst_hbm.at[p], dst_v)
            pltpu.sync_copy(w_hbm.at[p], w_v)
            lax.fori_loop(0, k_chunks, chunk, 0)

        if feat_split:
            def panel_loop(i, _):
                do_panel(sid * 2 + i)
                return 0

            lax.fori_loop(0, 2, panel_loop, 0)
        else:
            do_panel(sid * 2 + cid)

        plsc.subcore_barrier()

        for q in range(n_init):
            r = row0 + q * _LANES
            pltpu.sync_copy(sh.at[pl.ds(r, _LANES)], rows)
            pltpu.sync_copy(rows, out_hbm.at[cid].at[pl.ds(r, _LANES)])

    return agg_kernel


def _mm1_body(x_ref, w_ref, degp_ref, h3_ref, dis_ref):
    deg = jnp.sum(degp_ref[...], axis=0) + 1.0
    dis = jnp.where(deg > 0, lax.rsqrt(deg), 0.0)
    h = jnp.dot(x_ref[...], w_ref[...], preferred_element_type=jnp.float32)
    hh = dis[:, None] * h
    half = h3_ref.shape[2]
    h3_ref[0] = hh[:, :half]
    h3_ref[1] = hh[:, half:]
    dis_ref[...] = dis


def _mm2_body(a_ref, h3_ref, dis_ref, b1_ref, w2_ref, out_ref):
    dis = dis_ref[...]
    a = jnp.concatenate([a_ref[0] + h3_ref[0], a_ref[1] + h3_ref[1]], axis=1)
    z = jnp.maximum(dis[:, None] * a + b1_ref[...][None, :], 0.0)
    h2 = jnp.dot(z, w2_ref[...], preferred_element_type=jnp.float32)
    out_ref[...] = dis[:, None] * h2


def _end_body(p_ref, h2_ref, dis_ref, b2_ref, out_ref):
    s = p_ref[0] + p_ref[1] + h2_ref[...]
    out_ref[...] = dis_ref[...][:, None] * s + b2_ref[...][None, :]


def kernel(x, edge_index, edge_attr, W1, b1, W2, b2):
    n, in_ch = x.shape
    hid = W1.shape[1]
    out_ch = W2.shape[1]
    e = edge_attr.shape[0]
    assert hid == 2 * _LANES and out_ch == _LANES
    row_t = 1024
    n_pad = -(-n // row_t) * row_t

    src = edge_index[0].astype(jnp.int32)
    dst = edge_index[1].astype(jnp.int32)
    w = edge_attr.astype(jnp.float32)
    n_panels = 2 * _NS
    k_chunks = -(-e // (n_panels * _LANES))
    e_pad = n_panels * k_chunks * _LANES
    pad = e_pad - e
    srcp = jnp.pad(src, (0, pad)).reshape(n_panels, k_chunks, _LANES)
    dstp = jnp.pad(dst, (0, pad)).reshape(n_panels, k_chunks, _LANES)
    wp = jnp.pad(w, (0, pad)).reshape(n_panels, k_chunks, _LANES)

    deg_parts = _make_deg(n_pad, e_pad)(dstp.reshape(-1), wp.reshape(-1))

    xp = jnp.pad(x, ((0, n_pad - n), (0, 0)))
    grid = (n_pad // row_t,)
    h3, dis = pl.pallas_call(
        _mm1_body,
        grid=grid,
        in_specs=[
            pl.BlockSpec((row_t, in_ch), lambda r: (r, 0)),
            pl.BlockSpec((in_ch, hid), lambda r: (0, 0)),
            pl.BlockSpec((_NC * _NS, row_t), lambda r: (0, r)),
        ],
        out_specs=[
            pl.BlockSpec((2, row_t, _LANES), lambda r: (0, r, 0)),
            pl.BlockSpec((row_t,), lambda r: (r,)),
        ],
        out_shape=[
            jax.ShapeDtypeStruct((2, n_pad, _LANES), jnp.float32),
            jax.ShapeDtypeStruct((n_pad,), jnp.float32),
        ],
    )(xp, W1, deg_parts)

    agg1 = _make_agg(n_pad, k_chunks, True)(h3, srcp, dstp, wp)

    h2p = pl.pallas_call(
        _mm2_body,
        grid=grid,
        in_specs=[
            pl.BlockSpec((2, row_t, _LANES), lambda r: (0, r, 0)),
            pl.BlockSpec((2, row_t, _LANES), lambda r: (0, r, 0)),
            pl.BlockSpec((row_t,), lambda r: (r,)),
            pl.BlockSpec((hid,), lambda r: (0,)),
            pl.BlockSpec((hid, out_ch), lambda r: (0, 0)),
        ],
        out_specs=pl.BlockSpec((row_t, out_ch), lambda r: (r, 0)),
        out_shape=jax.ShapeDtypeStruct((n_pad, out_ch), jnp.float32),
    )(agg1, h3, dis, b1, W2)

    parts = _make_agg(n_pad, k_chunks, False)(h2p, srcp, dstp, wp)

    out = pl.pallas_call(
        _end_body,
        grid=grid,
        in_specs=[
            pl.BlockSpec((2, row_t, _LANES), lambda r: (0, r, 0)),
            pl.BlockSpec((row_t, out_ch), lambda r: (r, 0)),
            pl.BlockSpec((row_t,), lambda r: (r,)),
            pl.BlockSpec((out_ch,), lambda r: (0,)),
        ],
        out_specs=pl.BlockSpec((row_t, out_ch), lambda r: (r, 0)),
        out_shape=jax.ShapeDtypeStruct((n_pad, out_ch), jnp.float32),
    )(parts, h2p, dis, b2)
    return out[:n]


# speedup vs baseline: 1.1944x; 1.0056x over previous
import functools

import jax
import jax.numpy as jnp
from jax import lax
from jax.experimental import pallas as pl
from jax.experimental.pallas import tpu as pltpu
from jax.experimental.pallas import tpu_sc as plsc

_SC_PARAMS = pltpu.CompilerParams(needs_layout_passes=False)
_NC = 2
_NS = 16
_LANES = 128
_NSEM = 16


def _mesh():
    return plsc.VectorSubcoreMesh(core_axis_name="c", subcore_axis_name="s")


def _make_deg(n_acc, e_pad):
    per = e_pad // (_NC * _NS)
    n_vec = per // 16

    @functools.partial(
        pl.kernel,
        out_type=jax.ShapeDtypeStruct((_NC * _NS, n_acc), jnp.float32),
        mesh=_mesh(),
        scratch_types=[
            pltpu.VMEM((per,), jnp.int32),
            pltpu.VMEM((per,), jnp.float32),
            pltpu.VMEM((n_acc,), jnp.float32),
        ],
        compiler_params=_SC_PARAMS,
    )
    def deg_kernel(dst_hbm, w_hbm, out_hbm, dst_v, w_v, acc):
        cid = lax.axis_index("c")
        sid = lax.axis_index("s")
        wid = sid * _NC + cid
        base = wid * per
        pltpu.sync_copy(dst_hbm.at[pl.ds(base, per)], dst_v)
        pltpu.sync_copy(w_hbm.at[pl.ds(base, per)], w_v)

        zeros16 = jnp.zeros((16,), jnp.float32)

        def zero(i, _):
            acc[pl.ds(i * 16, 16)] = zeros16
            return 0

        lax.fori_loop(0, n_acc // 16, zero, 0)

        def add(i, _):
            idx16 = dst_v[pl.ds(i * 16, 16)]
            w16 = w_v[pl.ds(i * 16, 16)]
            plsc.addupdate_scatter(acc, [idx16], w16)
            return 0

        lax.fori_loop(0, n_vec, add, 0)
        pltpu.sync_copy(acc, out_hbm.at[wid])

    return deg_kernel


def _make_agg(n_pad, k_chunks, feat_split):
    rows_per_sub = n_pad // _NS
    assert rows_per_sub % _LANES == 0
    n_init = rows_per_sub // _LANES
    qrows = _LANES // _NSEM

    @functools.partial(
        pl.kernel,
        out_type=jax.ShapeDtypeStruct((2, n_pad, _LANES), jnp.float32),
        mesh=_mesh(),
        scratch_types=[
            pltpu.VMEM((k_chunks, _LANES), jnp.int32),
            pltpu.VMEM((k_chunks, _LANES), jnp.int32),
            pltpu.VMEM((k_chunks, _LANES), jnp.float32),
            pltpu.VMEM((_LANES, _LANES), jnp.float32),
            pltpu.VMEM_SHARED((n_pad, _LANES), jnp.float32),
            [pltpu.SemaphoreType.DMA] * _NSEM,
        ],
        compiler_params=_SC_PARAMS,
    )
    def agg_kernel(h_hbm, src_hbm, dst_hbm, w_hbm, out_hbm,
                   src_v, dst_v, w_v, rows, sh, sems):
        cid = lax.axis_index("c")
        sid = lax.axis_index("s")
        zeros16 = jnp.zeros((16,), jnp.float32)

        def h_view():
            return h_hbm.at[cid] if feat_split else h_hbm

        row0 = sid * rows_per_sub

        def zero_rows(i, _):
            for j in range(_LANES // 16):
                rows[i, pl.ds(j * 16, 16)] = zeros16
            return 0

        lax.fori_loop(0, _LANES, zero_rows, 0)
        for q in range(n_init):
            pltpu.sync_copy(rows, sh.at[pl.ds(row0 + q * _LANES, _LANES)])

        plsc.subcore_barrier()

        def chunk(k, _):
            for j in range(_NSEM):
                pltpu.async_copy(
                    h_view().at[src_v.at[k, pl.ds(j * qrows, qrows)]],
                    rows.at[pl.ds(j * qrows, qrows)], sems[j])
            for j in range(_NSEM):
                pltpu.make_async_copy(
                    h_view().at[src_v.at[k, pl.ds(j * qrows, qrows)]],
                    rows.at[pl.ds(j * qrows, qrows)], sems[j]).wait()
                for g in range(qrows // 16):
                    w16 = w_v[k, pl.ds(j * qrows + g * 16, 16)]
                    for l in range(16):
                        wi = w16[l]
                        i = j * qrows + g * 16 + l
                        for f in range(_LANES // 16):
                            rows[i, pl.ds(f * 16, 16)] = rows[i, pl.ds(f * 16, 16)] * wi
            pltpu.sync_copy(rows, sh.at[dst_v.at[k]], add=True)
            return 0

        def do_panel(p):
            pltpu.sync_copy(src_hbm.at[p], src_v)
            pltpu.sync_copy(dst_hbm.at[p], dst_v)
            pltpu.sync_copy(w_hbm.at[p], w_v)
            lax.fori_loop(0, k_chunks, chunk, 0)

        if feat_split:
            def panel_loop(i, _):
                do_panel(sid * 2 + i)
                return 0

            lax.fori_loop(0, 2, panel_loop, 0)
        else:
            do_panel(sid * 2 + cid)

        plsc.subcore_barrier()

        for q in range(n_init):
            r = row0 + q * _LANES
            pltpu.sync_copy(sh.at[pl.ds(r, _LANES)], rows)
            pltpu.sync_copy(rows, out_hbm.at[cid].at[pl.ds(r, _LANES)])

    return agg_kernel


def _mm1_body(x_ref, w_ref, degp_ref, h3_ref, dis_ref):
    deg = jnp.sum(degp_ref[...], axis=0) + 1.0
    dis = jnp.where(deg > 0, lax.rsqrt(deg), 0.0)
    h = jnp.dot(x_ref[...], w_ref[...], preferred_element_type=jnp.float32)
    hh = dis[:, None] * h
    half = h3_ref.shape[2]
    h3_ref[0] = hh[:, :half]
    h3_ref[1] = hh[:, half:]
    dis_ref[...] = dis


def _mm2_body(a_ref, h3_ref, dis_ref, b1_ref, w2_ref, out_ref):
    dis = dis_ref[...]
    a = jnp.concatenate([a_ref[0] + h3_ref[0], a_ref[1] + h3_ref[1]], axis=1)
    z = jnp.maximum(dis[:, None] * a + b1_ref[...][None, :], 0.0)
    h2 = jnp.dot(z, w2_ref[...], preferred_element_type=jnp.float32)
    out_ref[...] = dis[:, None] * h2


def _end_body(p_ref, h2_ref, dis_ref, b2_ref, out_ref):
    s = p_ref[0] + p_ref[1] + h2_ref[...]
    out_ref[...] = dis_ref[...][:, None] * s + b2_ref[...][None, :]


def kernel(x, edge_index, edge_attr, W1, b1, W2, b2):
    n, in_ch = x.shape
    hid = W1.shape[1]
    out_ch = W2.shape[1]
    e = edge_attr.shape[0]
    assert hid == 2 * _LANES and out_ch == _LANES
    row_t = 1024
    n_pad = -(-n // row_t) * row_t

    src = edge_index[0].astype(jnp.int32)
    dst = edge_index[1].astype(jnp.int32)
    w = edge_attr.astype(jnp.float32)
    n_panels = 2 * _NS
    k_chunks = -(-e // (n_panels * _LANES))
    e_pad = n_panels * k_chunks * _LANES
    pad = e_pad - e
    srcp = jnp.pad(src, (0, pad)).reshape(n_panels, k_chunks, _LANES)
    dstp = jnp.pad(dst, (0, pad)).reshape(n_panels, k_chunks, _LANES)
    wp = jnp.pad(w, (0, pad)).reshape(n_panels, k_chunks, _LANES)

    deg_parts = _make_deg(n_pad, e_pad)(dstp.reshape(-1), wp.reshape(-1))

    xp = jnp.pad(x, ((0, n_pad - n), (0, 0)))
    grid = (n_pad // row_t,)
    h3, dis = pl.pallas_call(
        _mm1_body,
        grid=grid,
        in_specs=[
            pl.BlockSpec((row_t, in_ch), lambda r: (r, 0)),
            pl.BlockSpec((in_ch, hid), lambda r: (0, 0)),
            pl.BlockSpec((_NC * _NS, row_t), lambda r: (0, r)),
        ],
        out_specs=[
            pl.BlockSpec((2, row_t, _LANES), lambda r: (0, r, 0)),
            pl.BlockSpec((row_t,), lambda r: (r,)),
        ],
        out_shape=[
            jax.ShapeDtypeStruct((2, n_pad, _LANES), jnp.float32),
            jax.ShapeDtypeStruct((n_pad,), jnp.float32),
        ],
    )(xp, W1, deg_parts)

    agg1 = _make_agg(n_pad, k_chunks, True)(h3, srcp, dstp, wp)

    h2p = pl.pallas_call(
        _mm2_body,
        grid=grid,
        in_specs=[
            pl.BlockSpec((2, row_t, _LANES), lambda r: (0, r, 0)),
            pl.BlockSpec((2, row_t, _LANES), lambda r: (0, r, 0)),
            pl.BlockSpec((row_t,), lambda r: (r,)),
            pl.BlockSpec((hid,), lambda r: (0,)),
            pl.BlockSpec((hid, out_ch), lambda r: (0, 0)),
        ],
        out_specs=pl.BlockSpec((row_t, out_ch), lambda r: (r, 0)),
        out_shape=jax.ShapeDtypeStruct((n_pad, out_ch), jnp.float32),
    )(agg1, h3, dis, b1, W2)

    parts = _make_agg(n_pad, k_chunks, False)(h2p, srcp, dstp, wp)

    out = pl.pallas_call(
        _end_body,
        grid=grid,
        in_specs=[
            pl.BlockSpec((2, row_t, _LANES), lambda r: (0, r, 0)),
            pl.BlockSpec((row_t, out_ch), lambda r: (r, 0)),
            pl.BlockSpec((row_t,), lambda r: (r,)),
            pl.BlockSpec((out_ch,), lambda r: (0,)),
        ],
        out_specs=pl.BlockSpec((row_t, out_ch), lambda r: (r, 0)),
        out_shape=jax.ShapeDtypeStruct((n_pad, out_ch), jnp.float32),
    )(parts, h2p, dis, b2)
    return out[:n]
